# Initial kernel scaffold; baseline (speedup 1.0000x reference)
import jax
import jax.numpy as jnp
from jax.experimental import pallas as pl


def kernel(indices):
    raise NotImplementedError("write your pallas kernel here")


# speedup vs baseline: 1.0390x; 1.0390x over previous
import functools

import jax
import jax.numpy as jnp
from jax import lax
from jax.experimental import pallas as pl
from jax.experimental.pallas import tpu as pltpu
from jax.experimental.pallas import tpu_sc as plsc

_B = 1024
_L = 200
_C = 100000
_W = _C // 4
_WPAD = 25008
_LP = 224
_NVEC = _LP // 16
_NW = 32
_RPW = _B // _NW
_NIDX = 112


def _sc_multihot(idx_pad):
    mesh = plsc.VectorSubcoreMesh(core_axis_name="c", subcore_axis_name="s")

    @functools.partial(
        pl.kernel,
        out_type=jax.ShapeDtypeStruct((_B * _W,), jnp.int32),
        mesh=mesh,
        compiler_params=pltpu.CompilerParams(needs_layout_passes=False),
        scratch_types=[
            pltpu.VMEM((_RPW * _LP,), jnp.int32),
            pltpu.VMEM((_WPAD,), jnp.int32),
            pltpu.VMEM((_WPAD,), jnp.int32),
            pltpu.VMEM((_LP,), jnp.int32),
            pltpu.VMEM((2, _NIDX), jnp.int32),
            pltpu.SemaphoreType.DMA,
            pltpu.SemaphoreType.DMA,
        ],
    )
    def k(idx_hbm, out_words, idx_v, zbuf, acc, outdata, outidx, zsem, ssem):
        wid = lax.axis_index("s") * 2 + lax.axis_index("c")
        base = wid * _RPW

        zeros16 = jnp.zeros((16,), jnp.int32)

        def zbody(i, carry):
            zbuf[pl.ds(i * 16, 16)] = zeros16
            acc[pl.ds(i * 16, 16)] = zeros16
            return carry

        lax.fori_loop(0, _WPAD // 16, zbody, 0)

        pltpu.sync_copy(idx_hbm.at[pl.ds(base * _LP, _RPW * _LP)], idx_v)

        zcopies = [
            pltpu.async_copy(
                zbuf.at[pl.ds(0, _W)],
                out_words.at[pl.ds((base + r) * _W, _W)],
                zsem,
            )
            for r in range(_RPW)
        ]
        for c in zcopies:
            c.wait()

        def row_body(r, carry):
            row = base + r
            for j in range(_NVEC):
                iv = idx_v[pl.ds(r * _LP + j * 16, 16)]
                w = lax.shift_right_logical(iv, 2)
                v = lax.shift_left(
                    jnp.ones((16,), jnp.int32),
                    lax.shift_left(lax.bitwise_and(iv, 3), 3),
                )
                plsc.addupdate_scatter(acc, [w], v)
            for j in range(_NVEC):
                iv = idx_v[pl.ds(r * _LP + j * 16, 16)]
                w = lax.shift_right_logical(iv, 2)
                g = plsc.load_gather(acc, [w])
                t = lax.bitwise_or(
                    g, lax.bitwise_and(lax.shift_right_logical(g, 1), 0x7F7F7F7F)
                )
                t = lax.bitwise_or(
                    t, lax.bitwise_and(lax.shift_right_logical(t, 2), 0x3F3F3F3F)
                )
                t = lax.bitwise_or(
                    t, lax.bitwise_and(lax.shift_right_logical(t, 4), 0x0F0F0F0F)
                )
                outdata[pl.ds(j * 16, 16)] = lax.bitwise_and(t, 0x01010101)
                outidx[j // 7, pl.ds((j % 7) * 16, 16)] = row * _W + w
            for j in range(_NVEC):
                iv = idx_v[pl.ds(r * _LP + j * 16, 16)]
                w = lax.shift_right_logical(iv, 2)
                plsc.store_scatter(acc, [w], zeros16)
            pltpu.async_copy(
                outdata.at[pl.ds(0, _NIDX)],
                out_words.at[outidx.at[0]],
                ssem,
            ).wait()
            pltpu.async_copy(
                outdata.at[pl.ds(_NIDX, _NIDX)],
                out_words.at[outidx.at[1]],
                ssem,
            ).wait()
            return carry

        lax.fori_loop(0, _RPW, row_body, 0)

    return k(idx_pad)


def kernel(indices):
    indices = indices.astype(jnp.int32)
    idx_pad = jnp.concatenate([indices, indices[:, : _LP - _L]], axis=1).reshape(-1)
    out_words = _sc_multihot(idx_pad)
    out_u8 = jax.lax.bitcast_convert_type(out_words, jnp.uint8)
    return out_u8.reshape(_B, _C).astype(jnp.bool_)


# speedup vs baseline: 1.0913x; 1.0504x over previous
import functools

import jax
import jax.numpy as jnp
from jax import lax
from jax.experimental import pallas as pl
from jax.experimental.pallas import tpu as pltpu
from jax.experimental.pallas import tpu_sc as plsc

_B = 1024
_L = 200
_C = 100000
_W = _C // 4
_WPAD = 25008
_LP = 224
_NVEC = _LP // 16
_NW = 32
_RPW = _B // _NW
_NIDX = 112


def _sc_multihot(idx_pad):
    mesh = plsc.VectorSubcoreMesh(core_axis_name="c", subcore_axis_name="s")

    @functools.partial(
        pl.kernel,
        out_type=jax.ShapeDtypeStruct((_B * _W,), jnp.int32),
        mesh=mesh,
        compiler_params=pltpu.CompilerParams(needs_layout_passes=False),
        scratch_types=[
            pltpu.VMEM((_RPW * _LP,), jnp.int32),
            pltpu.VMEM((_WPAD,), jnp.int32),
            pltpu.VMEM((_WPAD,), jnp.int32),
            pltpu.VMEM((_LP,), jnp.int32),
            pltpu.VMEM((2, _NIDX), jnp.int32),
            pltpu.SemaphoreType.DMA,
            pltpu.SemaphoreType.DMA,
        ],
    )
    def k(idx_hbm, out_words, idx_v, zbuf, acc, outdata, outidx, zsem, ssem):
        wid = lax.axis_index("s") * 2 + lax.axis_index("c")
        base = wid * _RPW

        zeros16 = jnp.zeros((16,), jnp.int32)

        def zbody(i, carry):
            zbuf[pl.ds(i * 16, 16)] = zeros16
            acc[pl.ds(i * 16, 16)] = zeros16
            return carry

        lax.fori_loop(0, _WPAD // 16, zbody, 0)

        pltpu.sync_copy(idx_hbm.at[pl.ds(base * _LP, _RPW * _LP)], idx_v)

        zcopies = [
            pltpu.async_copy(
                zbuf.at[pl.ds(0, _W)],
                out_words.at[pl.ds((base + r) * _W, _W)],
                zsem,
            )
            for r in range(_RPW)
        ]
        for c in zcopies:
            c.wait()

        def row_body(r, carry):
            row = base + r
            for j in range(_NVEC):
                iv = idx_v[pl.ds(r * _LP + j * 16, 16)]
                w = lax.shift_right_logical(iv, 2)
                v = lax.shift_left(
                    jnp.ones((16,), jnp.int32),
                    lax.shift_left(lax.bitwise_and(iv, 3), 3),
                )
                plsc.addupdate_scatter(acc, [w], v)
            for j in range(_NVEC):
                iv = idx_v[pl.ds(r * _LP + j * 16, 16)]
                w = lax.shift_right_logical(iv, 2)
                g = plsc.load_gather(acc, [w])
                t = lax.bitwise_or(
                    g, lax.bitwise_and(lax.shift_right_logical(g, 1), 0x7F7F7F7F)
                )
                t = lax.bitwise_or(
                    t, lax.bitwise_and(lax.shift_right_logical(t, 2), 0x3F3F3F3F)
                )
                t = lax.bitwise_or(
                    t, lax.bitwise_and(lax.shift_right_logical(t, 4), 0x0F0F0F0F)
                )
                outdata[pl.ds(j * 16, 16)] = lax.bitwise_and(t, 0x01010101)
                outidx[j // 7, pl.ds((j % 7) * 16, 16)] = row * _W + w
            for j in range(_NVEC):
                iv = idx_v[pl.ds(r * _LP + j * 16, 16)]
                w = lax.shift_right_logical(iv, 2)
                plsc.store_scatter(acc, [w], zeros16)
            pltpu.async_copy(
                outdata.at[pl.ds(0, _NIDX)],
                out_words.at[outidx.at[0]],
                ssem,
            ).wait()
            pltpu.async_copy(
                outdata.at[pl.ds(_NIDX, _NIDX)],
                out_words.at[outidx.at[1]],
                ssem,
            ).wait()
            return carry

        lax.fori_loop(0, _RPW, row_body, 0)

    return k(idx_pad)


def kernel(indices):
    indices = indices.astype(jnp.int32)
    idx_pad = jnp.concatenate([indices, indices[:, : _LP - _L]], axis=1).reshape(-1)
    out_words = _sc_multihot(idx_pad)
    shifts = jnp.arange(0, 32, 8, dtype=jnp.int32)
    bits = lax.shift_right_logical(
        out_words.reshape(_B, _W)[:, :, None], shifts[None, None, :]
    )
    return (lax.bitwise_and(bits, 1) != 0).reshape(_B, _C)
